# Initial kernel scaffold; baseline (speedup 1.0000x reference)
import jax
import jax.numpy as jnp
from jax.experimental import pallas as pl


def kernel(h_2, W_enc, b_enc, W_dec, b_dec):
    raise NotImplementedError("write your pallas kernel here")


# speedup vs baseline: 20.8964x; 20.8964x over previous
import functools

import jax
import jax.numpy as jnp
from jax.experimental import pallas as pl

INPUT_DIM = 768
OUTPUT_DIM = 768
LATENT_DIM = 4096
TOPK = 128
GAMMA = 1.0
BETA = 1.0

BLOCK_M = 256


def _body(h_ref, we_ref, be_ref, wd_ref, bd_ref, out_ref, zs_ref):
    h = h_ref[...]
    pre = jnp.dot(h, we_ref[...], preferred_element_type=jnp.float32)
    pre = pre + be_ref[...]
    z = jnp.maximum(pre, 0.0) + jnp.where(pre > GAMMA, BETA, 0.0)

    zb = jax.lax.bitcast_convert_type(z, jnp.int32)
    t = jnp.zeros((z.shape[0], 1), jnp.int32)
    for b in range(30, -1, -1):
        cand = t | (1 << b)
        cnt = jnp.sum((zb >= cand).astype(jnp.int32), axis=1, keepdims=True)
        t = jnp.where(cnt >= TOPK, cand, t)
    thr = jax.lax.bitcast_convert_type(t, jnp.float32)

    zs = jnp.where(z >= thr, z, 0.0)
    zs_ref[...] = zs
    out_ref[...] = (
        jnp.dot(zs, wd_ref[...], preferred_element_type=jnp.float32)
        + bd_ref[...]
    )


@jax.jit
def kernel(h_2, W_enc, b_enc, W_dec, b_dec):
    n = h_2.shape[0]
    grid = (n // BLOCK_M,)
    be = b_enc.reshape(1, LATENT_DIM)
    bd = b_dec.reshape(1, OUTPUT_DIM)
    h_1_recon, z_sparse = pl.pallas_call(
        _body,
        grid=grid,
        in_specs=[
            pl.BlockSpec((BLOCK_M, INPUT_DIM), lambda i: (i, 0)),
            pl.BlockSpec((INPUT_DIM, LATENT_DIM), lambda i: (0, 0)),
            pl.BlockSpec((1, LATENT_DIM), lambda i: (0, 0)),
            pl.BlockSpec((LATENT_DIM, OUTPUT_DIM), lambda i: (0, 0)),
            pl.BlockSpec((1, OUTPUT_DIM), lambda i: (0, 0)),
        ],
        out_specs=[
            pl.BlockSpec((BLOCK_M, OUTPUT_DIM), lambda i: (i, 0)),
            pl.BlockSpec((BLOCK_M, LATENT_DIM), lambda i: (i, 0)),
        ],
        out_shape=[
            jax.ShapeDtypeStruct((n, OUTPUT_DIM), jnp.float32),
            jax.ShapeDtypeStruct((n, LATENT_DIM), jnp.float32),
        ],
    )(h_2, W_enc, be, W_dec, bd)
    return (h_1_recon, z_sparse)


# speedup vs baseline: 42.3799x; 2.0281x over previous
import jax
import jax.numpy as jnp
from jax.experimental import pallas as pl

INPUT_DIM = 768
OUTPUT_DIM = 768
LATENT_DIM = 4096
TOPK = 128
GAMMA = 1.0
BETA = 1.0

BLOCK_M = 256
NPACK = 32
NGROUP = LATENT_DIM // NPACK


def _bit_transpose32(w):
    w = list(w)
    masks = {16: 0x0000FFFF, 8: 0x00FF00FF, 4: 0x0F0F0F0F,
             2: 0x33333333, 1: 0x55555555}
    for j in (16, 8, 4, 2, 1):
        m = jnp.int32(masks[j])
        k = 0
        while k < 32:
            a, b = w[k], w[k + j]
            t = (jax.lax.shift_right_logical(b, jnp.int32(j)) ^ a) & m
            w[k] = a ^ t
            w[k + j] = b ^ jax.lax.shift_left(t, jnp.int32(j))
            k = (k + j + 1) & ~j
    return [w[31 - b] for b in range(32)]


def _body(h_ref, we_ref, be_ref, wd_ref, bd_ref, out_ref, zs_ref):
    h = h_ref[...]
    pre = jnp.dot(h, we_ref[...], preferred_element_type=jnp.float32)
    pre = pre + be_ref[...]
    z = jnp.maximum(pre, 0.0) + jnp.where(pre > GAMMA, BETA, 0.0)

    m_rows = z.shape[0]
    zb = jax.lax.bitcast_convert_type(z, jnp.int32)
    slices = [zb[:, i * NGROUP:(i + 1) * NGROUP] for i in range(NPACK)]
    planes = _bit_transpose32(slices)

    active = jnp.full((m_rows, NGROUP), -1, jnp.int32)
    cnt_above = jnp.zeros((m_rows, 1), jnp.int32)
    t = jnp.zeros((m_rows, 1), jnp.int32)
    for b in range(30, -1, -1):
        ones = active & planes[b]
        n1 = jnp.sum(jax.lax.population_count(ones), axis=1, keepdims=True)
        take = (cnt_above + n1) >= TOPK
        t = jnp.where(take, t | (1 << b), t)
        active = jnp.where(take, ones, active ^ ones)
        cnt_above = jnp.where(take, cnt_above, cnt_above + n1)
    thr = jax.lax.bitcast_convert_type(t, jnp.float32)

    zs = jnp.where(z >= thr, z, 0.0)
    zs_ref[...] = zs
    out_ref[...] = (
        jnp.dot(zs, wd_ref[...], preferred_element_type=jnp.float32)
        + bd_ref[...]
    )


@jax.jit
def kernel(h_2, W_enc, b_enc, W_dec, b_dec):
    n = h_2.shape[0]
    grid = (n // BLOCK_M,)
    be = b_enc.reshape(1, LATENT_DIM)
    bd = b_dec.reshape(1, OUTPUT_DIM)
    h_1_recon, z_sparse = pl.pallas_call(
        _body,
        grid=grid,
        in_specs=[
            pl.BlockSpec((BLOCK_M, INPUT_DIM), lambda i: (i, 0)),
            pl.BlockSpec((INPUT_DIM, LATENT_DIM), lambda i: (0, 0)),
            pl.BlockSpec((1, LATENT_DIM), lambda i: (0, 0)),
            pl.BlockSpec((LATENT_DIM, OUTPUT_DIM), lambda i: (0, 0)),
            pl.BlockSpec((1, OUTPUT_DIM), lambda i: (0, 0)),
        ],
        out_specs=[
            pl.BlockSpec((BLOCK_M, OUTPUT_DIM), lambda i: (i, 0)),
            pl.BlockSpec((BLOCK_M, LATENT_DIM), lambda i: (i, 0)),
        ],
        out_shape=[
            jax.ShapeDtypeStruct((n, OUTPUT_DIM), jnp.float32),
            jax.ShapeDtypeStruct((n, LATENT_DIM), jnp.float32),
        ],
    )(h_2, W_enc, be, W_dec, bd)
    return (h_1_recon, z_sparse)
